# Initial kernel scaffold; baseline (speedup 1.0000x reference)
import jax
import jax.numpy as jnp
from jax.experimental import pallas as pl


def kernel(input_sequence, initial_state_h, initial_state_c, embedding_table, lstm_kernel, lstm_recurrent, lstm_bias):
    raise NotImplementedError("write your pallas kernel here")


# speedup vs baseline: 1.2402x; 1.2402x over previous
import functools

import jax
import jax.numpy as jnp
from jax import lax
from jax.experimental import pallas as pl
from jax.experimental.pallas import tpu as pltpu
from jax.experimental.pallas import tpu_sc as plsc


@functools.lru_cache(maxsize=None)
def _make_sc_gather(total_rows: int, d: int):
    info = plsc.get_sparse_core_info()
    nw = info.num_cores * info.num_subcores
    assert total_rows % nw == 0
    per_w = total_rows // nw
    chunk = 1600
    while per_w % chunk:
        chunk //= 2
    n_chunks = per_w // chunk

    mesh = plsc.VectorSubcoreMesh(core_axis_name="c", subcore_axis_name="s")

    @functools.partial(
        pl.kernel,
        mesh=mesh,
        out_type=jax.ShapeDtypeStruct((total_rows, d), jnp.float32),
        scratch_types=[
            pltpu.VMEM((chunk,), jnp.int32),
            pltpu.VMEM((chunk, d), jnp.float32),
            pltpu.SemaphoreType.DMA,
        ],
        compiler_params=pltpu.CompilerParams(use_tc_tiling_on_sc=False),
    )
    def gather_kernel(table_hbm, idx_hbm, out_hbm, idx_v, rows_v, sem):
        wid = lax.axis_index("s") * info.num_cores + lax.axis_index("c")

        def body(ci, carry):
            base = wid * per_w + ci * chunk
            pltpu.sync_copy(idx_hbm.at[pl.ds(base, chunk)], idx_v)
            pltpu.async_copy(table_hbm.at[idx_v], rows_v, sem).wait()
            pltpu.sync_copy(rows_v, out_hbm.at[pl.ds(base, chunk)])
            return carry

        lax.fori_loop(0, n_chunks, body, 0)

    return gather_kernel


def _lstm_body(ids_ref, emb_ref, h0_ref, c0_ref, wk_ref, wr_ref, b_ref,
               dec_ref, hf_ref, cf_ref, h_s, c_s, o_s,
               *, seq_len: int, t_chunk: int, units: int):
    tc = pl.program_id(1)

    @pl.when(tc == 0)
    def _():
        h_s[...] = h0_ref[...]
        c_s[...] = c0_ref[...]
        o_s[...] = jnp.zeros_like(o_s)

    h = h_s[...]
    c = c_s[...]
    out_prev = o_s[...]
    wk = wk_ref[...]
    wr = wr_ref[...]
    bias = b_ref[...]
    u = units
    m_all = (ids_ref[...] != 0).astype(jnp.float32)
    t_iota = lax.broadcasted_iota(jnp.int32, (seq_len, 1), 0)
    for t in range(t_chunk):
        x = emb_ref[t]
        z = (jnp.dot(x, wk, preferred_element_type=jnp.float32)
             + jnp.dot(h, wr, preferred_element_type=jnp.float32)
             + bias)
        gi = jax.nn.sigmoid(z[:, :u])
        gf = jax.nn.sigmoid(z[:, u:2 * u])
        gg = jnp.tanh(z[:, 2 * u:3 * u])
        go = jax.nn.sigmoid(z[:, 3 * u:])
        c_new = gf * c + gi * gg
        h_new = go * jnp.tanh(c_new)
        onehot = (t_iota == tc * t_chunk + t).astype(jnp.float32)
        m = jnp.dot(m_all, onehot, preferred_element_type=jnp.float32)
        km = 1.0 - m
        h = m * h_new + km * h
        c = m * c_new + km * c
        out_prev = m * h_new + km * out_prev
        dec_ref[:, t * u:(t + 1) * u] = out_prev
    h_s[...] = h
    c_s[...] = c
    o_s[...] = out_prev
    hf_ref[...] = h
    cf_ref[...] = c


def _run_lstm(ids, emb, h0, c0, wk, wr, bias, *,
              block_b: int = 512, t_chunk: int = 10):
    b, l = ids.shape
    d = emb.shape[-1]
    u = h0.shape[-1]
    grid = (b // block_b, l // t_chunk)
    body = functools.partial(_lstm_body, seq_len=l, t_chunk=t_chunk, units=u)
    dec, hf, cf = pl.pallas_call(
        body,
        grid=grid,
        in_specs=[
            pl.BlockSpec((block_b, l), lambda i, t: (i, 0)),
            pl.BlockSpec((t_chunk, block_b, d), lambda i, t: (t, i, 0)),
            pl.BlockSpec((block_b, u), lambda i, t: (i, 0)),
            pl.BlockSpec((block_b, u), lambda i, t: (i, 0)),
            pl.BlockSpec((d, 4 * u), lambda i, t: (0, 0)),
            pl.BlockSpec((u, 4 * u), lambda i, t: (0, 0)),
            pl.BlockSpec((1, 4 * u), lambda i, t: (0, 0)),
        ],
        out_specs=[
            pl.BlockSpec((block_b, t_chunk * u), lambda i, t: (i, t)),
            pl.BlockSpec((block_b, u), lambda i, t: (i, 0)),
            pl.BlockSpec((block_b, u), lambda i, t: (i, 0)),
        ],
        out_shape=[
            jax.ShapeDtypeStruct((b, l * u), jnp.float32),
            jax.ShapeDtypeStruct((b, u), jnp.float32),
            jax.ShapeDtypeStruct((b, u), jnp.float32),
        ],
        scratch_shapes=[
            pltpu.VMEM((block_b, u), jnp.float32),
            pltpu.VMEM((block_b, u), jnp.float32),
            pltpu.VMEM((block_b, u), jnp.float32),
        ],
        compiler_params=pltpu.CompilerParams(
            dimension_semantics=("arbitrary", "arbitrary"),
        ),
    )(ids, emb, h0, c0, wk, wr, bias)
    return dec.reshape(b, l, u), hf, cf


def kernel(input_sequence, initial_state_h, initial_state_c, embedding_table,
           lstm_kernel, lstm_recurrent, lstm_bias):
    b, l = input_sequence.shape
    v, d = embedding_table.shape
    u = initial_state_h.shape[-1]

    ids = input_sequence.astype(jnp.int32)
    idx_t = ids.T.reshape(-1)
    emb_flat = _make_sc_gather(l * b, d)(embedding_table, idx_t)
    emb = emb_flat.reshape(l, b, d)

    dec, h_fin, c_fin = _run_lstm(
        ids, emb, initial_state_h, initial_state_c,
        lstm_kernel, lstm_recurrent, lstm_bias.reshape(1, 4 * u))
    return dec, h_fin, c_fin


# speedup vs baseline: 1.2594x; 1.0154x over previous
import functools

import jax
import jax.numpy as jnp
from jax import lax
from jax.experimental import pallas as pl
from jax.experimental.pallas import tpu as pltpu
from jax.experimental.pallas import tpu_sc as plsc


@functools.lru_cache(maxsize=None)
def _make_sc_gather(total_rows: int, d: int):
    info = plsc.get_sparse_core_info()
    nw = info.num_cores * info.num_subcores
    assert total_rows % nw == 0
    per_w = total_rows // nw
    chunk = 1600
    while per_w % chunk:
        chunk //= 2
    n_chunks = per_w // chunk

    mesh = plsc.VectorSubcoreMesh(core_axis_name="c", subcore_axis_name="s")

    @functools.partial(
        pl.kernel,
        mesh=mesh,
        out_type=jax.ShapeDtypeStruct((total_rows, d), jnp.float32),
        scratch_types=[
            pltpu.VMEM((chunk,), jnp.int32),
            pltpu.VMEM((chunk, d), jnp.float32),
            pltpu.SemaphoreType.DMA,
        ],
        compiler_params=pltpu.CompilerParams(use_tc_tiling_on_sc=False),
    )
    def gather_kernel(table_hbm, idx_hbm, out_hbm, idx_v, rows_v, sem):
        wid = lax.axis_index("s") * info.num_cores + lax.axis_index("c")

        def body(ci, carry):
            base = wid * per_w + ci * chunk
            pltpu.sync_copy(idx_hbm.at[pl.ds(base, chunk)], idx_v)
            pltpu.async_copy(table_hbm.at[idx_v], rows_v, sem).wait()
            pltpu.sync_copy(rows_v, out_hbm.at[pl.ds(base, chunk)])
            return carry

        lax.fori_loop(0, n_chunks, body, 0)

    return gather_kernel


def _lstm_body(ids_ref, emb_ref, h0_ref, c0_ref, wk_ref, wr_ref, b_ref,
               dec_ref, hf_ref, cf_ref, h_s, c_s, o_s,
               *, seq_len: int, t_chunk: int, units: int):
    tc = pl.program_id(1)

    @pl.when(tc == 0)
    def _():
        h_s[...] = h0_ref[...]
        c_s[...] = c0_ref[...]
        o_s[...] = jnp.zeros_like(o_s)

    h = h_s[...]
    c = c_s[...]
    out_prev = o_s[...]
    wk = wk_ref[...]
    wr = wr_ref[...]
    bias = b_ref[...]
    u = units
    m_all = (ids_ref[...] != 0).astype(jnp.float32)
    t_iota = lax.broadcasted_iota(jnp.int32, (seq_len, 1), 0)
    d = wk_ref.shape[0]
    for t in range(t_chunk):
        x = emb_ref[:, t * d:(t + 1) * d]
        z = (jnp.dot(x, wk, preferred_element_type=jnp.float32)
             + jnp.dot(h, wr, preferred_element_type=jnp.float32)
             + bias)
        gi = jax.nn.sigmoid(z[:, :u])
        gf = jax.nn.sigmoid(z[:, u:2 * u])
        gg = jnp.tanh(z[:, 2 * u:3 * u])
        go = jax.nn.sigmoid(z[:, 3 * u:])
        c_new = gf * c + gi * gg
        h_new = go * jnp.tanh(c_new)
        onehot = (t_iota == tc * t_chunk + t).astype(jnp.float32)
        m = jnp.dot(m_all, onehot, preferred_element_type=jnp.float32)
        km = 1.0 - m
        h = m * h_new + km * h
        c = m * c_new + km * c
        out_prev = m * h_new + km * out_prev
        dec_ref[:, t * u:(t + 1) * u] = out_prev
    h_s[...] = h
    c_s[...] = c
    o_s[...] = out_prev
    hf_ref[...] = h
    cf_ref[...] = c


def _run_lstm(ids, emb, h0, c0, wk, wr, bias, *,
              block_b: int = 512, t_chunk: int = 10):
    b, l = ids.shape
    d = emb.shape[-1] // l
    u = h0.shape[-1]
    grid = (b // block_b, l // t_chunk)
    body = functools.partial(_lstm_body, seq_len=l, t_chunk=t_chunk, units=u)
    dec, hf, cf = pl.pallas_call(
        body,
        grid=grid,
        in_specs=[
            pl.BlockSpec((block_b, l), lambda i, t: (i, 0)),
            pl.BlockSpec((block_b, t_chunk * d), lambda i, t: (i, t)),
            pl.BlockSpec((block_b, u), lambda i, t: (i, 0)),
            pl.BlockSpec((block_b, u), lambda i, t: (i, 0)),
            pl.BlockSpec((d, 4 * u), lambda i, t: (0, 0)),
            pl.BlockSpec((u, 4 * u), lambda i, t: (0, 0)),
            pl.BlockSpec((1, 4 * u), lambda i, t: (0, 0)),
        ],
        out_specs=[
            pl.BlockSpec((block_b, t_chunk * u), lambda i, t: (i, t)),
            pl.BlockSpec((block_b, u), lambda i, t: (i, 0)),
            pl.BlockSpec((block_b, u), lambda i, t: (i, 0)),
        ],
        out_shape=[
            jax.ShapeDtypeStruct((b, l * u), jnp.float32),
            jax.ShapeDtypeStruct((b, u), jnp.float32),
            jax.ShapeDtypeStruct((b, u), jnp.float32),
        ],
        scratch_shapes=[
            pltpu.VMEM((block_b, u), jnp.float32),
            pltpu.VMEM((block_b, u), jnp.float32),
            pltpu.VMEM((block_b, u), jnp.float32),
        ],
        compiler_params=pltpu.CompilerParams(
            dimension_semantics=("arbitrary", "arbitrary"),
        ),
    )(ids, emb, h0, c0, wk, wr, bias)
    return dec.reshape(b, l, u), hf, cf


def kernel(input_sequence, initial_state_h, initial_state_c, embedding_table,
           lstm_kernel, lstm_recurrent, lstm_bias):
    b, l = input_sequence.shape
    v, d = embedding_table.shape
    u = initial_state_h.shape[-1]

    ids = input_sequence.astype(jnp.int32)
    idx = ids.reshape(-1)
    emb_flat = _make_sc_gather(b * l, d)(embedding_table, idx)
    emb = emb_flat.reshape(b, l * d)

    dec, h_fin, c_fin = _run_lstm(
        ids, emb, initial_state_h, initial_state_c,
        lstm_kernel, lstm_recurrent, lstm_bias.reshape(1, 4 * u))
    return dec, h_fin, c_fin


# speedup vs baseline: 1.2620x; 1.0021x over previous
import functools

import jax
import jax.numpy as jnp
from jax import lax
from jax.experimental import pallas as pl
from jax.experimental.pallas import tpu as pltpu
from jax.experimental.pallas import tpu_sc as plsc


@functools.lru_cache(maxsize=None)
def _make_sc_gather(total_rows: int, d: int):
    info = plsc.get_sparse_core_info()
    nw = info.num_cores * info.num_subcores
    assert total_rows % nw == 0
    per_w = total_rows // nw
    chunk = 1600
    while per_w % chunk:
        chunk //= 2
    n_chunks = per_w // chunk

    mesh = plsc.VectorSubcoreMesh(core_axis_name="c", subcore_axis_name="s")

    @functools.partial(
        pl.kernel,
        mesh=mesh,
        out_type=jax.ShapeDtypeStruct((total_rows, d), jnp.float32),
        scratch_types=[
            pltpu.VMEM((chunk,), jnp.int32),
            pltpu.VMEM((chunk, d), jnp.float32),
            pltpu.SemaphoreType.DMA,
        ],
        compiler_params=pltpu.CompilerParams(use_tc_tiling_on_sc=False),
    )
    def gather_kernel(table_hbm, idx_hbm, out_hbm, idx_v, rows_v, sem):
        wid = lax.axis_index("s") * info.num_cores + lax.axis_index("c")

        def body(ci, carry):
            base = wid * per_w + ci * chunk
            pltpu.sync_copy(idx_hbm.at[pl.ds(base, chunk)], idx_v)
            pltpu.async_copy(table_hbm.at[idx_v], rows_v, sem).wait()
            pltpu.sync_copy(rows_v, out_hbm.at[pl.ds(base, chunk)])
            return carry

        lax.fori_loop(0, n_chunks, body, 0)

    return gather_kernel


def _lstm_body(ids_ref, emb_ref, h0_ref, c0_ref, wk_ref, wr_ref, b_ref,
               dec_ref, hf_ref, cf_ref, h_s, c_s, o_s,
               *, seq_len: int, t_chunk: int, units: int):
    tc = pl.program_id(1)

    @pl.when(tc == 0)
    def _():
        h_s[...] = h0_ref[...]
        c_s[...] = c0_ref[...]
        o_s[...] = jnp.zeros_like(o_s)

    h = h_s[...]
    c = c_s[...]
    out_prev = o_s[...]
    wk = wk_ref[...]
    wr = wr_ref[...]
    bias = b_ref[...]
    u = units
    m_all = (ids_ref[...] != 0).astype(jnp.float32)
    t_iota = lax.broadcasted_iota(jnp.int32, (seq_len, 1), 0)
    d = wk_ref.shape[0]
    for t in range(t_chunk):
        onehot_x = (t_iota == tc * t_chunk + t).astype(jnp.float32)
        x = emb_ref[:, t * d:(t + 1) * d]
        z = (jnp.dot(x, wk, preferred_element_type=jnp.float32)
             + jnp.dot(h, wr, preferred_element_type=jnp.float32)
             + bias)
        gi = jax.nn.sigmoid(z[:, :u])
        gf = jax.nn.sigmoid(z[:, u:2 * u])
        gg = jnp.tanh(z[:, 2 * u:3 * u])
        go = jax.nn.sigmoid(z[:, 3 * u:])
        c_new = gf * c + gi * gg
        h_new = go * jnp.tanh(c_new)
        m = jnp.dot(m_all, onehot_x, preferred_element_type=jnp.float32)
        km = 1.0 - m
        h = m * h_new + km * h
        c = m * c_new + km * c
        out_prev = m * h_new + km * out_prev
        dec_ref[:, t * u:(t + 1) * u] = out_prev
    h_s[...] = h
    c_s[...] = c
    o_s[...] = out_prev
    hf_ref[...] = h
    cf_ref[...] = c


def _run_lstm(ids, emb, h0, c0, wk, wr, bias, *,
              block_b: int = 512, t_chunk: int = 10):
    b, l = ids.shape
    d = emb.shape[-1] // l
    u = h0.shape[-1]
    grid = (b // block_b, l // t_chunk)
    body = functools.partial(_lstm_body, seq_len=l, t_chunk=t_chunk, units=u)
    dec, hf, cf = pl.pallas_call(
        body,
        grid=grid,
        in_specs=[
            pl.BlockSpec((block_b, l), lambda i, t: (i, 0)),
            pl.BlockSpec((block_b, t_chunk * d), lambda i, t: (i, t)),
            pl.BlockSpec((block_b, u), lambda i, t: (i, 0)),
            pl.BlockSpec((block_b, u), lambda i, t: (i, 0)),
            pl.BlockSpec((d, 4 * u), lambda i, t: (0, 0)),
            pl.BlockSpec((u, 4 * u), lambda i, t: (0, 0)),
            pl.BlockSpec((1, 4 * u), lambda i, t: (0, 0)),
        ],
        out_specs=[
            pl.BlockSpec((block_b, t_chunk * u), lambda i, t: (i, t)),
            pl.BlockSpec((block_b, u), lambda i, t: (i, 0)),
            pl.BlockSpec((block_b, u), lambda i, t: (i, 0)),
        ],
        out_shape=[
            jax.ShapeDtypeStruct((b, l * u), jnp.float32),
            jax.ShapeDtypeStruct((b, u), jnp.float32),
            jax.ShapeDtypeStruct((b, u), jnp.float32),
        ],
        scratch_shapes=[
            pltpu.VMEM((block_b, u), jnp.float32),
            pltpu.VMEM((block_b, u), jnp.float32),
            pltpu.VMEM((block_b, u), jnp.float32),
        ],
        compiler_params=pltpu.CompilerParams(
            dimension_semantics=("arbitrary", "arbitrary"),
        ),
    )(ids, emb, h0, c0, wk, wr, bias)
    return dec.reshape(b, l, u), hf, cf


def kernel(input_sequence, initial_state_h, initial_state_c, embedding_table,
           lstm_kernel, lstm_recurrent, lstm_bias):
    b, l = input_sequence.shape
    v, d = embedding_table.shape
    u = initial_state_h.shape[-1]

    ids = input_sequence.astype(jnp.int32)
    idx = ids.reshape(-1)
    emb_flat = _make_sc_gather(b * l, d)(embedding_table, idx)
    emb = emb_flat.reshape(b, l * d)

    dec, h_fin, c_fin = _run_lstm(
        ids, emb, initial_state_h, initial_state_c,
        lstm_kernel, lstm_recurrent, lstm_bias.reshape(1, 4 * u))
    return dec, h_fin, c_fin
